# baseline (speedup 1.0000x reference)
import jax
import jax.numpy as jnp
from jax import lax
from jax.experimental import pallas as pl
from jax.experimental.pallas import tpu as pltpu

BG, MD, KL, EL, DL, RL = 5, 4, 3, 3, 3, 4
DK, DE, DD, DR = BG * MD * KL, BG * MD * EL, BG * DL, RL
NROW = 101


def _sel(col, stride, width, n_lanes, permuted=False):
    n_sub = stride
    j_b = lax.broadcasted_iota(jnp.int32, (width, n_lanes), 0)
    b_b = lax.broadcasted_iota(jnp.int32, (width, n_lanes), 1)
    masked = jnp.where(j_b // n_sub == b_b,
                       jnp.broadcast_to(col, (width, n_lanes)), 0.0)
    j_a = lax.broadcasted_iota(jnp.int32, (n_sub, width, n_lanes), 1)
    p_a = lax.broadcasted_iota(jnp.int32, (n_sub, width, n_lanes), 0)
    off = KL * (p_a % MD) + p_a // MD if permuted else p_a
    t = jnp.where(j_a % n_sub == off, masked[None, :, :], 0.0)
    return jnp.sum(t, axis=1)


def _body(x_ref, wk_hbm, we_hbm, wd_hbm, wr_hbm,
          ok_ref, oe_ref, od_ref, or_ref,
          wk_ref, we_ref, wd_ref, wr_ref, sem):
    cps = [
        pltpu.make_async_copy(wk_hbm, wk_ref, sem),
        pltpu.make_async_copy(we_hbm, we_ref, sem),
        pltpu.make_async_copy(wd_hbm, wd_ref, sem),
        pltpu.make_async_copy(wr_hbm, wr_ref, sem),
    ]
    for cp in cps:
        cp.start()
    v100 = jnp.broadcast_to(x_ref[...], (1, NROW)) * 100.0
    i0 = v100.astype(jnp.int32)
    ivec = i0 - (i0.astype(jnp.float32) > v100).astype(jnp.int32)
    lmask = ivec == lax.broadcasted_iota(jnp.int32, (1, NROW), 1)
    for cp in cps:
        cp.wait()

    def col(ref, width):
        sel = jnp.where(jnp.broadcast_to(lmask, (width, NROW)), ref[...], 0.0)
        return jnp.sum(sel, axis=1, keepdims=True)

    ck = col(wk_ref, DK)
    ce = col(we_ref, DE)
    cd = col(wd_ref, DD)
    cr = col(wr_ref, DR)

    sk = _sel(ck, KL * MD, DK, BG, permuted=True)
    se = _sel(ce, EL * MD, DE, BG, permuted=True)
    for k in range(KL):
        ok_ref[k, :, :] = sk[k * MD:(k + 1) * MD, :]
        oe_ref[k, :, :] = se[k * MD:(k + 1) * MD, :]
    od_ref[...] = _sel(cd, DL, DD, BG)
    or_ref[...] = _sel(cr, 1, DR, RL)


@jax.jit
def _lookup(x, wkT, weT, wdT, wrT):
    return pl.pallas_call(
        _body,
        in_specs=[pl.BlockSpec(memory_space=pltpu.VMEM)]
        + [pl.BlockSpec(memory_space=pl.ANY)] * 4,
        scratch_shapes=[
            pltpu.VMEM((DK, NROW), jnp.float32),
            pltpu.VMEM((DE, NROW), jnp.float32),
            pltpu.VMEM((DD, NROW), jnp.float32),
            pltpu.VMEM((DR, NROW), jnp.float32),
            pltpu.SemaphoreType.DMA,
        ],
        out_shape=(
            jax.ShapeDtypeStruct((KL, MD, BG), jnp.float32),
            jax.ShapeDtypeStruct((EL, MD, BG), jnp.float32),
            jax.ShapeDtypeStruct((DL, BG), jnp.float32),
            jax.ShapeDtypeStruct((1, RL), jnp.float32),
        ),
    )(x, wkT, weT, wdT, wrT)


def kernel(x, W_kernel, W_expand, W_depth, W_res):
    okT, oeT, odT, orr = _lookup(x, W_kernel.T, W_expand.T, W_depth.T, W_res.T)
    return (
        jnp.transpose(okT, (2, 1, 0)),
        jnp.transpose(oeT, (2, 1, 0)),
        jnp.transpose(odT, (1, 0)),
        orr,
    )


# speedup vs baseline: 1.0125x; 1.0125x over previous
import jax
import jax.numpy as jnp
from jax import lax
from jax.experimental import pallas as pl
from jax.experimental.pallas import tpu as pltpu

BG, MD, KL, EL, DL, RL = 5, 4, 3, 3, 3, 4
DK, DE, DD, DR = BG * MD * KL, BG * MD * EL, BG * DL, RL
NROW = 101


def _sel(col, stride, width, n_lanes, permuted=False):
    n_sub = stride
    j_b = lax.broadcasted_iota(jnp.int32, (width, n_lanes), 0)
    b_b = lax.broadcasted_iota(jnp.int32, (width, n_lanes), 1)
    masked = jnp.where(j_b // n_sub == b_b,
                       jnp.broadcast_to(col, (width, n_lanes)), 0.0)
    j_a = lax.broadcasted_iota(jnp.int32, (n_sub, width, n_lanes), 1)
    p_a = lax.broadcasted_iota(jnp.int32, (n_sub, width, n_lanes), 0)
    off = KL * (p_a % MD) + p_a // MD if permuted else p_a
    t = jnp.where(j_a % n_sub == off, masked[None, :, :], 0.0)
    return jnp.sum(t, axis=1)


def _body(x_hbm, wk_hbm, we_hbm, wd_hbm, wr_hbm,
          ok_ref, oe_ref, od_ref, or_ref,
          x_ref, wk_ref, we_ref, wd_ref, wr_ref, sem):
    xcp = pltpu.make_async_copy(x_hbm, x_ref, sem)
    xcp.start()
    cps = [
        pltpu.make_async_copy(wk_hbm, wk_ref, sem),
        pltpu.make_async_copy(we_hbm, we_ref, sem),
        pltpu.make_async_copy(wd_hbm, wd_ref, sem),
        pltpu.make_async_copy(wr_hbm, wr_ref, sem),
    ]
    for cp in cps:
        cp.start()
    xcp.wait()
    v100 = jnp.broadcast_to(x_ref[...], (1, NROW)) * 100.0
    i0 = v100.astype(jnp.int32)
    ivec = i0 - (i0.astype(jnp.float32) > v100).astype(jnp.int32)
    lmask = ivec == lax.broadcasted_iota(jnp.int32, (1, NROW), 1)
    for cp in cps:
        cp.wait()

    def col(ref, width):
        sel = jnp.where(jnp.broadcast_to(lmask, (width, NROW)), ref[...], 0.0)
        return jnp.sum(sel, axis=1, keepdims=True)

    ck = col(wk_ref, DK)
    ce = col(we_ref, DE)
    cd = col(wd_ref, DD)
    cr = col(wr_ref, DR)

    sk = _sel(ck, KL * MD, DK, BG, permuted=True)
    se = _sel(ce, EL * MD, DE, BG, permuted=True)
    for k in range(KL):
        ok_ref[k, :, :] = sk[k * MD:(k + 1) * MD, :]
        oe_ref[k, :, :] = se[k * MD:(k + 1) * MD, :]
    od_ref[...] = _sel(cd, DL, DD, BG)
    or_ref[...] = _sel(cr, 1, DR, RL)


@jax.jit
def _lookup(x, wkT, weT, wdT, wrT):
    return pl.pallas_call(
        _body,
        in_specs=[pl.BlockSpec(memory_space=pl.ANY)] * 5,
        scratch_shapes=[
            pltpu.VMEM((1, 1), jnp.float32),
            pltpu.VMEM((DK, NROW), jnp.float32),
            pltpu.VMEM((DE, NROW), jnp.float32),
            pltpu.VMEM((DD, NROW), jnp.float32),
            pltpu.VMEM((DR, NROW), jnp.float32),
            pltpu.SemaphoreType.DMA,
        ],
        out_shape=(
            jax.ShapeDtypeStruct((KL, MD, BG), jnp.float32),
            jax.ShapeDtypeStruct((EL, MD, BG), jnp.float32),
            jax.ShapeDtypeStruct((DL, BG), jnp.float32),
            jax.ShapeDtypeStruct((1, RL), jnp.float32),
        ),
    )(x, wkT, weT, wdT, wrT)


def kernel(x, W_kernel, W_expand, W_depth, W_res):
    okT, oeT, odT, orr = _lookup(x, W_kernel.T, W_expand.T, W_depth.T, W_res.T)
    return (
        jnp.transpose(okT, (2, 1, 0)),
        jnp.transpose(oeT, (2, 1, 0)),
        jnp.transpose(odT, (1, 0)),
        orr,
    )
